# initial kernel scaffold (unmeasured)
import jax
import jax.numpy as jnp
from jax import lax
from jax.experimental import pallas as pl
from jax.experimental.pallas import tpu as pltpu


def kernel(
    t,
):
    def body(*refs):
        pass

    out_shape = jax.ShapeDtypeStruct(..., jnp.float32)
    return pl.pallas_call(body, out_shape=out_shape)(...)


# baseline (device time: 54713 ns/iter reference)
import jax
import jax.numpy as jnp
from jax import lax
from jax.experimental import pallas as pl
from jax.experimental.pallas import tpu as pltpu

N_DEV = 4


def kernel(t):
    m, n = t.shape

    def body(x_ref, out_ref, recv_ref, acc_ref, send_sems, recv_sems):
        my = lax.axis_index("i")
        p1 = my ^ 1
        p2 = 3 - my

        barrier_sem = pltpu.get_barrier_semaphore()
        for nbr in (p1, p2):
            pl.semaphore_signal(
                barrier_sem, inc=1,
                device_id=(nbr,), device_id_type=pl.DeviceIdType.MESH,
            )
        pl.semaphore_wait(barrier_sem, 2)

        rdma1 = pltpu.make_async_remote_copy(
            src_ref=x_ref,
            dst_ref=recv_ref.at[0],
            send_sem=send_sems.at[0],
            recv_sem=recv_sems.at[0],
            device_id=(p1,),
            device_id_type=pl.DeviceIdType.MESH,
        )
        rdma1.start()
        rdma1.wait()
        acc_ref[...] = x_ref[...] + recv_ref[0]

        rdma2 = pltpu.make_async_remote_copy(
            src_ref=acc_ref,
            dst_ref=recv_ref.at[1],
            send_sem=send_sems.at[1],
            recv_sem=recv_sems.at[1],
            device_id=(p2,),
            device_id_type=pl.DeviceIdType.MESH,
        )
        rdma2.start()
        rdma2.wait()

        s = acc_ref[...] + recv_ref[1]
        r = jnp.maximum(s, 0.0)
        out_ref[...] = jnp.tanh(s) * s * s + r * r * r

    return pl.pallas_call(
        body,
        out_shape=jax.ShapeDtypeStruct((m, n), jnp.float32),
        in_specs=[pl.BlockSpec(memory_space=pltpu.VMEM)],
        out_specs=pl.BlockSpec(memory_space=pltpu.VMEM),
        scratch_shapes=[
            pltpu.VMEM((2, m, n), jnp.float32),
            pltpu.VMEM((m, n), jnp.float32),
            pltpu.SemaphoreType.DMA((2,)),
            pltpu.SemaphoreType.DMA((2,)),
        ],
        compiler_params=pltpu.CompilerParams(collective_id=0),
    )(t)


# device time: 32578 ns/iter; 1.6794x vs baseline; 1.6794x over previous
import jax
import jax.numpy as jnp
from jax import lax
from jax.experimental import pallas as pl
from jax.experimental.pallas import tpu as pltpu

N_DEV = 4
QROWS = 256


def kernel(t):
    m, n = t.shape

    def body(x_ref, out_ref, rs_ref, send_rs, recv_rs, send_ag, recv_ag):
        my = lax.axis_index("i")

        barrier_sem = pltpu.get_barrier_semaphore()
        for r in range(1, N_DEV):
            pl.semaphore_signal(
                barrier_sem, inc=1,
                device_id=((my + r) % N_DEV,),
                device_id_type=pl.DeviceIdType.MESH,
            )
        pl.semaphore_wait(barrier_sem, N_DEV - 1)

        rs = []
        for r in range(1, N_DEV):
            o = (my + r) % N_DEV
            rdma = pltpu.make_async_remote_copy(
                src_ref=x_ref.at[pl.ds(o * QROWS, QROWS), :],
                dst_ref=rs_ref.at[r - 1],
                send_sem=send_rs.at[r - 1],
                recv_sem=recv_rs.at[r - 1],
                device_id=(o,),
                device_id_type=pl.DeviceIdType.MESH,
            )
            rdma.start()
            rs.append(rdma)
        for rdma in rs:
            rdma.wait_recv()

        s = (
            x_ref[pl.ds(my * QROWS, QROWS), :]
            + rs_ref[0]
            + rs_ref[1]
            + rs_ref[2]
        )
        relu = jnp.maximum(s, 0.0)
        out_ref[pl.ds(my * QROWS, QROWS), :] = (
            jnp.tanh(s) * s * s + relu * relu * relu
        )

        ag = []
        for r in range(1, N_DEV):
            o = (my + r) % N_DEV
            rdma = pltpu.make_async_remote_copy(
                src_ref=out_ref.at[pl.ds(my * QROWS, QROWS), :],
                dst_ref=out_ref.at[pl.ds(my * QROWS, QROWS), :],
                send_sem=send_ag.at[r - 1],
                recv_sem=recv_ag.at[r - 1],
                device_id=(o,),
                device_id_type=pl.DeviceIdType.MESH,
            )
            rdma.start()
            ag.append(rdma)
        for rdma in ag:
            rdma.wait_recv()

        for rdma in rs:
            rdma.wait_send()
        for rdma in ag:
            rdma.wait_send()

    return pl.pallas_call(
        body,
        out_shape=jax.ShapeDtypeStruct((m, n), jnp.float32),
        in_specs=[pl.BlockSpec(memory_space=pltpu.VMEM)],
        out_specs=pl.BlockSpec(memory_space=pltpu.VMEM),
        scratch_shapes=[
            pltpu.VMEM((N_DEV - 1, QROWS, n), jnp.float32),
            pltpu.SemaphoreType.DMA((N_DEV - 1,)),
            pltpu.SemaphoreType.DMA((N_DEV - 1,)),
            pltpu.SemaphoreType.DMA((N_DEV - 1,)),
            pltpu.SemaphoreType.DMA((N_DEV - 1,)),
        ],
        compiler_params=pltpu.CompilerParams(collective_id=0),
    )(t)


# device time: 27143 ns/iter; 2.0157x vs baseline; 1.2002x over previous
import jax
import jax.numpy as jnp
from jax import lax
from jax.experimental import pallas as pl
from jax.experimental.pallas import tpu as pltpu

N_DEV = 4
HROWS = 512
CCOLS = 256


def kernel(t):
    m, n = t.shape

    def body(x_ref, out_ref, ra1, rb1, ra2, rb2, acc_a, acc_b,
             send_sems, recv_sems):
        my = lax.axis_index("i")
        u = my ^ 1
        v = 3 - my

        h_a = (my ^ (my >> 1)) & 1
        h_b = (my >> 1) & 1

        barrier_sem = pltpu.get_barrier_semaphore()
        for nbr in (u, v):
            pl.semaphore_signal(
                barrier_sem, inc=1,
                device_id=(nbr,), device_id_type=pl.DeviceIdType.MESH,
            )
        pl.semaphore_wait(barrier_sem, 2)

        def exch(src, dst, sem, peer):
            return pltpu.make_async_remote_copy(
                src_ref=src, dst_ref=dst,
                send_sem=send_sems.at[sem], recv_sem=recv_sems.at[sem],
                device_id=(peer,), device_id_type=pl.DeviceIdType.MESH,
            )

        a1 = exch(x_ref.at[pl.ds((1 - h_a) * HROWS, HROWS), pl.ds(0, CCOLS)],
                  ra1, 0, u)
        b1 = exch(x_ref.at[pl.ds((1 - h_b) * HROWS, HROWS), pl.ds(CCOLS, CCOLS)],
                  rb1, 1, v)
        a1.start()
        b1.start()

        a1.wait_recv()
        acc_a[...] = x_ref[pl.ds(h_a * HROWS, HROWS), pl.ds(0, CCOLS)] + ra1[...]
        a2 = exch(acc_a, ra2, 2, v)
        a2.start()

        b1.wait_recv()
        acc_b[...] = x_ref[pl.ds(h_b * HROWS, HROWS), pl.ds(CCOLS, CCOLS)] + rb1[...]
        b2 = exch(acc_b, rb2, 3, u)
        b2.start()

        a2.wait_recv()
        s_a = acc_a[...] + ra2[...]
        r_a = jnp.maximum(s_a, 0.0)
        out_ref[pl.ds(h_a * HROWS, HROWS), pl.ds(0, CCOLS)] = (
            jnp.tanh(s_a) * s_a * s_a + r_a * r_a * r_a
        )
        ag_a = exch(out_ref.at[pl.ds(h_a * HROWS, HROWS), pl.ds(0, CCOLS)],
                    out_ref.at[pl.ds(h_a * HROWS, HROWS), pl.ds(0, CCOLS)],
                    4, u)
        ag_a.start()

        b2.wait_recv()
        s_b = acc_b[...] + rb2[...]
        r_b = jnp.maximum(s_b, 0.0)
        out_ref[pl.ds(h_b * HROWS, HROWS), pl.ds(CCOLS, CCOLS)] = (
            jnp.tanh(s_b) * s_b * s_b + r_b * r_b * r_b
        )
        ag_b = exch(out_ref.at[pl.ds(h_b * HROWS, HROWS), pl.ds(CCOLS, CCOLS)],
                    out_ref.at[pl.ds(h_b * HROWS, HROWS), pl.ds(CCOLS, CCOLS)],
                    5, v)
        ag_b.start()

        ag_a.wait_recv()
        ag_b.wait_recv()

        for rdma in (a1, b1, a2, b2, ag_a, ag_b):
            rdma.wait_send()

    return pl.pallas_call(
        body,
        out_shape=jax.ShapeDtypeStruct((m, n), jnp.float32),
        in_specs=[pl.BlockSpec(memory_space=pltpu.VMEM)],
        out_specs=pl.BlockSpec(memory_space=pltpu.VMEM),
        scratch_shapes=[
            pltpu.VMEM((HROWS, CCOLS), jnp.float32),
            pltpu.VMEM((HROWS, CCOLS), jnp.float32),
            pltpu.VMEM((HROWS, CCOLS), jnp.float32),
            pltpu.VMEM((HROWS, CCOLS), jnp.float32),
            pltpu.VMEM((HROWS, CCOLS), jnp.float32),
            pltpu.VMEM((HROWS, CCOLS), jnp.float32),
            pltpu.SemaphoreType.DMA((6,)),
            pltpu.SemaphoreType.DMA((6,)),
        ],
        compiler_params=pltpu.CompilerParams(collective_id=0),
    )(t)
